# baseline (speedup 1.0000x reference)
import functools

import jax
import jax.numpy as jnp
from jax import lax
from jax.experimental import pallas as pl
from jax.experimental.pallas import tpu as pltpu
from jax.experimental.pallas import tpu_sc as plsc

S = 4096
B = 128
K = 136
BR = 1024
BC = 512
NB = S // BR
NQ = BC // 128
SCALE_R = 64.0
SCALE_C = 128.0
MATCH = 128.0 * SCALE_R * SCALE_C
NEG = -3e9


def _match_body(x_hbm, out_ref, sgn_r_ref, sgn_c_ref, x_vmem, dma_sem):
    i = pl.program_id(0)

    @pl.when(i == 0)
    def _precompute():
        pltpu.make_async_copy(x_hbm, x_vmem, dma_sem).start()
        pltpu.make_async_copy(x_hbm, x_vmem, dma_sem).wait()
        m = x_vmem[...] > 0.5
        lane = lax.broadcasted_iota(jnp.int32, (S, K - B), 1)
        pos = lax.broadcasted_iota(jnp.int32, (S, K - B), 0)
        sr = jnp.where(m, SCALE_R, -SCALE_R)
        sc = jnp.where(m, SCALE_C, -SCALE_C)
        ext_r = jnp.where(lane < 2, -1.0, 0.0)
        ext_c = jnp.where(lane == 0, (pos >> 7).astype(jnp.float32) * 128.0,
                          jnp.where(lane == 1,
                                    (pos & 127).astype(jnp.float32), 0.0))
        sgn_r_ref[...] = jnp.concatenate([sr, ext_r], axis=1).astype(jnp.bfloat16)
        sgn_c_ref[...] = jnp.concatenate([sc, ext_c], axis=1).astype(jnp.bfloat16)

    br = sgn_r_ref[pl.ds(i * BR, BR), :]

    def score_block(j):
        bcb = sgn_c_ref[pl.ds(j * BC, BC), :]
        return lax.dot_general(br, bcb, (((1,), (1,)), ((), ())),
                               preferred_element_type=jnp.float32)

    NS_ = BR // 128

    def body(j, carry):
        carry = list(carry)
        for n in range(BC // 256):
            bcn = sgn_c_ref[pl.ds(j * BC + n * 256, 256), :]
            for mm in range(BR // 256):
                brm = sgn_r_ref[pl.ds(i * BR + mm * 256, 256), :]
                d = lax.dot_general(brm, bcn, (((1,), (1,)), ((), ())),
                                    preferred_element_type=jnp.float32)
                for h in range(2):
                    m = 2 * mm + h
                    dm = d[h * 128:(h + 1) * 128, :]
                    carry[m] = jnp.maximum(carry[m], dm[:, 0:128])
                    carry[m] = jnp.maximum(carry[m], dm[:, 128:256])
        return tuple(carry)

    RBC = BR // BC
    init = tuple(jnp.full((128, 128), NEG, jnp.float32) for _ in range(NS_))
    carry = list(lax.fori_loop(0, i * RBC, body, init))

    tri = lax.broadcasted_iota(jnp.int32, (128, 128), 1) <= \
        lax.broadcasted_iota(jnp.int32, (128, 128), 0)
    for k in range(NS_):
        rows = BR - 128 * k
        brk = sgn_r_ref[pl.ds(i * BR + 128 * k, rows), :]
        bck = sgn_c_ref[pl.ds(i * BR + 128 * k, 128), :]
        d = lax.dot_general(brk, bck, (((1,), (1,)), ((), ())),
                            preferred_element_type=jnp.float32)
        carry[k] = jnp.maximum(carry[k], jnp.where(tri, d[:128, :], NEG))
        for m in range(k + 1, NS_):
            dm = d[(m - k) * 128:(m - k) * 128 + 128, :]
            carry[m] = jnp.maximum(carry[m], dm)

    for m in range(NS_):
        score = jnp.max(carry[m], axis=1)
        out_ref[0, 0, pl.ds(m * 128, 128)] = (MATCH - score).astype(jnp.int32)


def _best_indices(x, interpret=False):
    out = pl.pallas_call(
        _match_body,
        grid=(NB,),
        in_specs=[pl.BlockSpec(memory_space=pl.ANY)],
        out_specs=pl.BlockSpec((1, 1, BR), lambda i: (i, 0, 0)),
        out_shape=jax.ShapeDtypeStruct((NB, 1, BR), jnp.int32),
        scratch_shapes=[
            pltpu.VMEM((S, K), jnp.bfloat16),
            pltpu.VMEM((S, K), jnp.bfloat16),
            pltpu.VMEM((S, B), jnp.float32),
            pltpu.SemaphoreType.DMA,
        ],
        interpret=interpret,
    )(x)
    return out.reshape(S)


_NW = 32
_BPW = S // _NW


def _sc_gather(x, idx):
    mesh = plsc.VectorSubcoreMesh(core_axis_name="c", subcore_axis_name="s")

    @functools.partial(
        pl.kernel,
        out_type=jax.ShapeDtypeStruct((S, B), jnp.float32),
        mesh=mesh,
        scratch_types=[
            pltpu.VMEM((_BPW,), jnp.int32),
            pltpu.VMEM((_BPW, B), jnp.float32),
            pltpu.SemaphoreType.DMA,
        ],
    )
    def k(table_hbm, idx_hbm, out_hbm, idx_v, rows_v, sem):
        wid = lax.axis_index("s") * 2 + lax.axis_index("c")
        base = wid * _BPW
        pltpu.sync_copy(idx_hbm.at[pl.ds(base, _BPW)], idx_v)
        pltpu.async_copy(table_hbm.at[idx_v], rows_v, sem).wait()
        pltpu.sync_copy(rows_v, out_hbm.at[pl.ds(base, _BPW)])

    return k(x, idx)


def kernel(x):
    best = _best_indices(x)
    return _sc_gather(x, best)


# speedup vs baseline: 1.2416x; 1.2416x over previous
import functools

import jax
import jax.numpy as jnp
from jax import lax
from jax.experimental import pallas as pl
from jax.experimental.pallas import tpu as pltpu
from jax.experimental.pallas import tpu_sc as plsc

S = 4096
B = 128
K = 136
NCH = S // 128
NP = S // 256
SCALE_R = 64.0
SCALE_C = 128.0
MATCH = 128.0 * SCALE_R * SCALE_C
NEG = -3e9


def _match_body(x_hbm, out_ref, sgn_r_ref, sgn_c_ref, x_vmem, dma_sem):
    pltpu.make_async_copy(x_hbm, x_vmem, dma_sem).start()
    pltpu.make_async_copy(x_hbm, x_vmem, dma_sem).wait()
    m = x_vmem[...] > 0.5
    lane = lax.broadcasted_iota(jnp.int32, (S, K - B), 1)
    pos = lax.broadcasted_iota(jnp.int32, (S, K - B), 0)
    sr = jnp.where(m, SCALE_R, -SCALE_R)
    sc = jnp.where(m, SCALE_C, -SCALE_C)
    ext_r = jnp.where(lane < 2, -1.0, 0.0)
    ext_c = jnp.where(lane == 0, (pos >> 7).astype(jnp.float32) * 128.0,
                      jnp.where(lane == 1,
                                (pos & 127).astype(jnp.float32), 0.0))
    sgn_r_ref[...] = jnp.concatenate([sr, ext_r], axis=1).astype(jnp.bfloat16)
    sgn_c_ref[...] = jnp.concatenate([sc, ext_c], axis=1).astype(jnp.bfloat16)

    tri = lax.broadcasted_iota(jnp.int32, (128, 128), 1) <= \
        lax.broadcasted_iota(jnp.int32, (128, 128), 0)
    carry = [None] * NCH
    for p in range(NP):
        row0 = 256 * p
        brp = sgn_r_ref[pl.ds(row0, S - row0), :]
        bcp = sgn_c_ref[pl.ds(row0, 256), :]
        d = lax.dot_general(brp, bcp, (((1,), (1,)), ((), ())),
                            preferred_element_type=jnp.float32)
        c0 = 2 * p
        d00 = jnp.where(tri, d[0:128, 0:128], NEG)
        carry[c0] = d00 if carry[c0] is None else jnp.maximum(carry[c0], d00)
        v = jnp.maximum(d[128:256, 0:128],
                        jnp.where(tri, d[128:256, 128:256], NEG))
        carry[c0 + 1] = (v if carry[c0 + 1] is None
                         else jnp.maximum(carry[c0 + 1], v))
        for mm in range(c0 + 2, NCH):
            r = 128 * (mm - c0)
            dm = d[r:r + 128, :]
            v = jnp.maximum(dm[:, 0:128], dm[:, 128:256])
            carry[mm] = (v if carry[mm] is None
                         else jnp.maximum(carry[mm], v))

    for mm in range(NCH):
        score = jnp.max(carry[mm], axis=1)
        out_ref[0, 0, pl.ds(mm * 128, 128)] = (MATCH - score).astype(jnp.int32)


def _best_indices(x, interpret=False):
    out = pl.pallas_call(
        _match_body,
        grid=(1,),
        in_specs=[pl.BlockSpec(memory_space=pl.ANY)],
        out_specs=pl.BlockSpec((1, 1, S), lambda i: (0, 0, 0)),
        out_shape=jax.ShapeDtypeStruct((1, 1, S), jnp.int32),
        scratch_shapes=[
            pltpu.VMEM((S, K), jnp.bfloat16),
            pltpu.VMEM((S, K), jnp.bfloat16),
            pltpu.VMEM((S, B), jnp.float32),
            pltpu.SemaphoreType.DMA,
        ],
        interpret=interpret,
    )(x)
    return out.reshape(S)


_NW = 32
_BPW = S // _NW


def _sc_gather(x, idx):
    mesh = plsc.VectorSubcoreMesh(core_axis_name="c", subcore_axis_name="s")

    @functools.partial(
        pl.kernel,
        out_type=jax.ShapeDtypeStruct((S, B), jnp.float32),
        mesh=mesh,
        scratch_types=[
            pltpu.VMEM((_BPW,), jnp.int32),
            pltpu.VMEM((_BPW, B), jnp.float32),
            pltpu.SemaphoreType.DMA,
        ],
    )
    def k(table_hbm, idx_hbm, out_hbm, idx_v, rows_v, sem):
        wid = lax.axis_index("s") * 2 + lax.axis_index("c")
        base = wid * _BPW
        pltpu.sync_copy(idx_hbm.at[pl.ds(base, _BPW)], idx_v)
        pltpu.async_copy(table_hbm.at[idx_v], rows_v, sem).wait()
        pltpu.sync_copy(rows_v, out_hbm.at[pl.ds(base, _BPW)])

    return k(x, idx)


def kernel(x):
    best = _best_indices(x)
    return _sc_gather(x, best)
